# Initial kernel scaffold; baseline (speedup 1.0000x reference)
import jax
import jax.numpy as jnp
from jax.experimental import pallas as pl


def kernel(x, W):
    raise NotImplementedError("write your pallas kernel here")


# speedup vs baseline: 1.1040x; 1.1040x over previous
import jax
import jax.numpy as jnp
from jax import lax
from jax.experimental import pallas as pl
from jax.experimental.pallas import tpu as pltpu
from jax.experimental.pallas import tpu_sc as plsc

NUM_EMB = 1_000_000
D = 32
BATCH = 16384
HIST = 50
TOTAL = BATCH * HIST
NC = 2
NS = 16
NW = NC * NS
PER_W = TOTAL // NW
SUB = 128
GROUP = 8
ROWS_STEP = SUB * GROUP
NSTEP = PER_W // ROWS_STEP
NSUB = PER_W // SUB


def _body(x_hbm, w_hbm, out_hbm, idx_v, rows_v, sem):
    wid = lax.axis_index("s") * NC + lax.axis_index("c")
    pltpu.sync_copy(x_hbm.at[wid], idx_v)

    def step(i, carry):
        descs = []
        for j in range(GROUP):
            descs.append(
                pltpu.async_copy(
                    w_hbm.at[idx_v.at[i * GROUP + j]],
                    rows_v.at[pl.ds(j * SUB, SUB)],
                    sem,
                )
            )
        for d in descs:
            d.wait()
        pltpu.sync_copy(
            rows_v,
            out_hbm.at[pl.ds(wid * PER_W + i * ROWS_STEP, ROWS_STEP)],
        )
        return carry

    lax.fori_loop(0, NSTEP, step, 0)


def kernel(x, W):
    xf = x.reshape(NW, NSUB, SUB)
    mesh = plsc.VectorSubcoreMesh(core_axis_name="c", subcore_axis_name="s")
    out = pl.kernel(
        _body,
        out_type=jax.ShapeDtypeStruct((TOTAL, D), jnp.float32),
        mesh=mesh,
        compiler_params=pltpu.CompilerParams(use_tc_tiling_on_sc=False),
        scratch_types=[
            pltpu.VMEM((NSUB, SUB), jnp.int32),
            pltpu.VMEM((ROWS_STEP, D), jnp.float32),
            pltpu.SemaphoreType.DMA,
        ],
    )(xf, W)
    return out.reshape(BATCH, HIST, D)


# speedup vs baseline: 1.5439x; 1.3984x over previous
import jax
import jax.numpy as jnp
from jax import lax
from jax.experimental import pallas as pl
from jax.experimental.pallas import tpu as pltpu
from jax.experimental.pallas import tpu_sc as plsc

NUM_EMB = 1_000_000
D = 32
BATCH = 16384
HIST = 50
NC = 2
NS = 16
NW = NC * NS
BCHUNK = 1024
NBC = BATCH // BCHUNK
UNITS = HIST * NBC
PER_W = UNITS // NW
SUB = 128
GROUP = BCHUNK // SUB


def _body(xt_hbm, w_hbm, out_hbm, idx_v, rows_v, tr_v, gsem, wsem):
    wid = lax.axis_index("s") * NC + lax.axis_index("c")
    lane_bc = lax.broadcasted_iota(jnp.int32, (16,), 0) * BCHUNK

    def unit(u, carry):
        h = u // NBC
        bc = u % NBC
        pltpu.sync_copy(xt_hbm.at[h, pl.ds(bc * GROUP, GROUP)], idx_v)
        descs = [
            pltpu.async_copy(
                w_hbm.at[idx_v.at[j]],
                rows_v.at[pl.ds(j * SUB, SUB)],
                gsem,
            )
            for j in range(GROUP)
        ]
        for dsc in descs:
            dsc.wait()

        out_base = (h * D) * BATCH + bc * BCHUNK

        def row_block(r, c):
            for k in range(4):
                i = r * 4 + k
                lo = rows_v[i, pl.ds(0, 16)]
                hi = rows_v[i, pl.ds(16, 16)]
                plsc.store_scatter(tr_v, [lane_bc + i], lo)
                plsc.store_scatter(
                    tr_v, [lane_bc + (16 * BCHUNK + i)], hi
                )
            return c

        lax.fori_loop(0, BCHUNK // 4, row_block, 0)
        wdescs = [
            pltpu.async_copy(
                tr_v.at[pl.ds(f * BCHUNK, BCHUNK)],
                out_hbm.at[pl.ds(out_base + f * BATCH, BCHUNK)],
                wsem,
            )
            for f in range(D)
        ]
        for dsc in wdescs:
            dsc.wait()
        return carry

    lax.fori_loop(wid * PER_W, (wid + 1) * PER_W, unit, 0)


def kernel(x, W):
    xt = x.T.reshape(HIST, NBC * GROUP, SUB)
    mesh = plsc.VectorSubcoreMesh(core_axis_name="c", subcore_axis_name="s")
    out = pl.kernel(
        _body,
        out_type=jax.ShapeDtypeStruct((HIST * D * BATCH,), jnp.float32),
        mesh=mesh,
        compiler_params=pltpu.CompilerParams(
            use_tc_tiling_on_sc=False, needs_layout_passes=False
        ),
        scratch_types=[
            pltpu.VMEM((GROUP, SUB), jnp.int32),
            pltpu.VMEM((BCHUNK, D), jnp.float32),
            pltpu.VMEM((BCHUNK * D,), jnp.float32),
            pltpu.SemaphoreType.DMA,
            pltpu.SemaphoreType.DMA,
        ],
    )(xt, W)
    return out.reshape(HIST, D, BATCH).transpose(2, 0, 1)
